# baseline (speedup 1.0000x reference)
import functools

import jax
import jax.numpy as jnp
from jax import lax
from jax.experimental import pallas as pl
from jax.experimental.pallas import tpu as pltpu, tpu_sc as plsc

D_IN = 32
D_OUT = 32
NUM_CORES = 2
NUM_SUBCORES = 16
NUM_WORKERS = NUM_CORES * NUM_SUBCORES
LANES = 16

CHUNK = 32
N_BUF = 2


def _make_kernel(n_rows: int):
    assert n_rows % (NUM_WORKERS * CHUNK * N_BUF) == 0
    b_per_w = n_rows // NUM_WORKERS
    n_chunks = b_per_w // CHUNK
    mesh = plsc.VectorSubcoreMesh(
        core_axis_name="c", subcore_axis_name="s",
        num_cores=NUM_CORES, num_subcores=NUM_SUBCORES)

    @functools.partial(
        pl.kernel,
        out_type=jax.ShapeDtypeStruct((n_rows * D_OUT,), jnp.float32),
        mesh=mesh,
        compiler_params=pltpu.CompilerParams(needs_layout_passes=False),
        scratch_types=[
            pltpu.VMEM((b_per_w,), jnp.int32),
            pltpu.VMEM((N_BUF, CHUNK, D_IN * D_OUT), jnp.float32),
            pltpu.VMEM((N_BUF, CHUNK, D_IN), jnp.float32),
            pltpu.VMEM((N_BUF, CHUNK * D_OUT), jnp.float32),
            pltpu.SemaphoreType.DMA((N_BUF,)),
            pltpu.SemaphoreType.DMA((N_BUF,)),
        ],
    )
    def seg_poly(w_hbm, x_hbm, idx_hbm, out_hbm,
                 idx_v, w_v, x_v, o_v, sem_w, sem_x):
        wid = lax.axis_index("s") * NUM_CORES + lax.axis_index("c")
        base = wid * b_per_w

        pltpu.sync_copy(idx_hbm.at[pl.ds(base, b_per_w)], idx_v)

        def issue(k, b):
            row0 = base + k * CHUNK
            pltpu.async_copy(w_hbm.at[idx_v.at[pl.ds(k * CHUNK, CHUNK)]],
                             w_v.at[b], sem_w.at[b])
            pltpu.async_copy(x_hbm.at[pl.ds(row0, CHUNK), :],
                             x_v.at[b], sem_x.at[b])

        def compute(k, b):
            row0 = base + k * CHUNK
            pltpu.make_async_copy(
                w_hbm.at[idx_v.at[pl.ds(k * CHUNK, CHUNK)]],
                w_v.at[b], sem_w.at[b]).wait()
            pltpu.make_async_copy(
                x_hbm.at[pl.ds(row0, CHUNK), :],
                x_v.at[b], sem_x.at[b]).wait()

            @plsc.parallel_loop(0, CHUNK, unroll=4)
            def row_body(r):
                xv0 = x_v[b, r, pl.ds(0, LANES)]
                xv1 = x_v[b, r, pl.ds(LANES, LANES)]
                acc0 = jnp.zeros((LANES,), jnp.float32)
                acc1 = jnp.zeros((LANES,), jnp.float32)
                for i in range(D_IN):
                    xs = xv0[i] if i < LANES else xv1[i - LANES]
                    xb = lax.broadcast(xs, (LANES,))
                    acc0 = acc0 + xb * w_v[b, r, pl.ds(i * D_OUT, LANES)]
                    acc1 = acc1 + xb * w_v[b, r,
                                           pl.ds(i * D_OUT + LANES, LANES)]
                o_v[b, pl.ds(r * D_OUT, LANES)] = acc0
                o_v[b, pl.ds(r * D_OUT + LANES, LANES)] = acc1

            pltpu.sync_copy(o_v.at[b],
                            out_hbm.at[pl.ds(row0 * D_OUT, CHUNK * D_OUT)])

        issue(0, 0)

        @pl.loop(0, n_chunks, step=N_BUF)
        def chunk_loop(k0):
            for b in range(N_BUF):
                k = k0 + b

                @pl.when(k + 1 < n_chunks)
                def _():
                    issue(k + 1, (b + 1) % N_BUF)

                compute(k, b)

    return seg_poly


@jax.jit
def kernel(weights, x, weight_indices):
    n_rows = x.shape[0]
    out = _make_kernel(n_rows)(weights, x, weight_indices)
    return out.reshape(n_rows, D_OUT)


# speedup vs baseline: 1.0592x; 1.0592x over previous
import functools

import jax
import jax.numpy as jnp
from jax import lax
from jax.experimental import pallas as pl
from jax.experimental.pallas import tpu as pltpu, tpu_sc as plsc

D_IN = 32
D_OUT = 32
NUM_CORES = 2
NUM_SUBCORES = 16
NUM_WORKERS = NUM_CORES * NUM_SUBCORES
LANES = 16

CHUNK = 32
N_BUF = 2


def _make_kernel(n_rows: int):
    assert n_rows % (NUM_WORKERS * CHUNK * N_BUF) == 0
    b_per_w = n_rows // NUM_WORKERS
    n_chunks = b_per_w // CHUNK
    mesh = plsc.VectorSubcoreMesh(
        core_axis_name="c", subcore_axis_name="s",
        num_cores=NUM_CORES, num_subcores=NUM_SUBCORES)

    @functools.partial(
        pl.kernel,
        out_type=jax.ShapeDtypeStruct((n_rows, D_OUT), jnp.float32),
        mesh=mesh,
        compiler_params=pltpu.CompilerParams(needs_layout_passes=False),
        scratch_types=[
            pltpu.VMEM((b_per_w,), jnp.int32),
            pltpu.VMEM((N_BUF, CHUNK, D_IN * D_OUT), jnp.float32),
            pltpu.VMEM((N_BUF, CHUNK, D_IN), jnp.float32),
            pltpu.VMEM((N_BUF, CHUNK, D_OUT), jnp.float32),
            pltpu.SemaphoreType.DMA((N_BUF,)),
            pltpu.SemaphoreType.DMA((N_BUF,)),
        ],
    )
    def seg_poly(w_hbm, x_hbm, idx_hbm, out_hbm,
                 idx_v, w_v, x_v, o_v, sem_w, sem_x):
        wid = lax.axis_index("s") * NUM_CORES + lax.axis_index("c")
        base = wid * b_per_w

        pltpu.sync_copy(idx_hbm.at[pl.ds(base, b_per_w)], idx_v)

        def issue(k, b):
            row0 = base + k * CHUNK
            pltpu.async_copy(w_hbm.at[idx_v.at[pl.ds(k * CHUNK, CHUNK)]],
                             w_v.at[b], sem_w.at[b])
            pltpu.async_copy(x_hbm.at[pl.ds(row0, CHUNK), :],
                             x_v.at[b], sem_x.at[b])

        def compute(k, b):
            row0 = base + k * CHUNK
            pltpu.make_async_copy(
                w_hbm.at[idx_v.at[pl.ds(k * CHUNK, CHUNK)]],
                w_v.at[b], sem_w.at[b]).wait()
            pltpu.make_async_copy(
                x_hbm.at[pl.ds(row0, CHUNK), :],
                x_v.at[b], sem_x.at[b]).wait()

            @plsc.parallel_loop(0, CHUNK, unroll=4)
            def row_body(r):
                xv0 = x_v[b, r, pl.ds(0, LANES)]
                xv1 = x_v[b, r, pl.ds(LANES, LANES)]
                acc0 = jnp.zeros((LANES,), jnp.float32)
                acc1 = jnp.zeros((LANES,), jnp.float32)
                for i in range(D_IN):
                    xs = xv0[i] if i < LANES else xv1[i - LANES]
                    xb = lax.broadcast(xs, (LANES,))
                    acc0 = acc0 + xb * w_v[b, r, pl.ds(i * D_OUT, LANES)]
                    acc1 = acc1 + xb * w_v[b, r,
                                           pl.ds(i * D_OUT + LANES, LANES)]
                o_v[b, r, pl.ds(0, LANES)] = acc0
                o_v[b, r, pl.ds(LANES, LANES)] = acc1

            pltpu.sync_copy(o_v.at[b], out_hbm.at[pl.ds(row0, CHUNK), :])

        issue(0, 0)

        @pl.loop(0, n_chunks, step=N_BUF)
        def chunk_loop(k0):
            for b in range(N_BUF):
                k = k0 + b

                @pl.when(k + 1 < n_chunks)
                def _():
                    issue(k + 1, (b + 1) % N_BUF)

                compute(k, b)

    return seg_poly


@jax.jit
def kernel(weights, x, weight_indices):
    return _make_kernel(x.shape[0])(weights, x, weight_indices)


# speedup vs baseline: 1.0778x; 1.0176x over previous
import functools

import jax
import jax.numpy as jnp
from jax import lax
from jax.experimental import pallas as pl
from jax.experimental.pallas import tpu as pltpu, tpu_sc as plsc

D_IN = 32
D_OUT = 32
NUM_CORES = 2
NUM_SUBCORES = 16
NUM_WORKERS = NUM_CORES * NUM_SUBCORES
LANES = 16

CHUNK = 32
N_BUF = 2


def _make_kernel(n_rows: int):
    assert n_rows % (NUM_WORKERS * CHUNK * N_BUF) == 0
    b_per_w = n_rows // NUM_WORKERS
    n_chunks = b_per_w // CHUNK
    mesh = plsc.VectorSubcoreMesh(
        core_axis_name="c", subcore_axis_name="s",
        num_cores=NUM_CORES, num_subcores=NUM_SUBCORES)

    @functools.partial(
        pl.kernel,
        out_type=jax.ShapeDtypeStruct((n_rows, D_OUT), jnp.float32),
        mesh=mesh,
        compiler_params=pltpu.CompilerParams(needs_layout_passes=False),
        scratch_types=[
            pltpu.VMEM((b_per_w,), jnp.int32),
            pltpu.VMEM((N_BUF, CHUNK, D_IN * D_OUT), jnp.float32),
            pltpu.VMEM((N_BUF, CHUNK, D_IN), jnp.float32),
            pltpu.VMEM((N_BUF, CHUNK, D_OUT), jnp.float32),
            pltpu.SemaphoreType.DMA((N_BUF,)),
            pltpu.SemaphoreType.DMA((N_BUF,)),
        ],
    )
    def seg_poly(w_hbm, x_hbm, idx_hbm, out_hbm,
                 idx_v, w_v, x_v, o_v, sem_w, sem_x):
        wid = lax.axis_index("s") * NUM_CORES + lax.axis_index("c")
        base = wid * b_per_w

        pltpu.sync_copy(idx_hbm.at[pl.ds(base, b_per_w)], idx_v)

        def issue(k, b):
            row0 = base + k * CHUNK
            pltpu.async_copy(w_hbm.at[idx_v.at[pl.ds(k * CHUNK, CHUNK)]],
                             w_v.at[b], sem_w.at[b])
            pltpu.async_copy(x_hbm.at[pl.ds(row0, CHUNK), :],
                             x_v.at[b], sem_x.at[b])

        def compute(k, b):
            row0 = base + k * CHUNK
            pltpu.make_async_copy(
                w_hbm.at[idx_v.at[pl.ds(k * CHUNK, CHUNK)]],
                w_v.at[b], sem_w.at[b]).wait()
            pltpu.make_async_copy(
                x_hbm.at[pl.ds(row0, CHUNK), :],
                x_v.at[b], sem_x.at[b]).wait()

            @plsc.parallel_loop(0, CHUNK, unroll=4)
            def row_body(r):
                xv0 = x_v[b, r, pl.ds(0, LANES)]
                xv1 = x_v[b, r, pl.ds(LANES, LANES)]
                acc0 = [None] * 4
                acc1 = [None] * 4
                for i in range(D_IN):
                    xs = xv0[i] if i < LANES else xv1[i - LANES]
                    xb = lax.broadcast(xs, (LANES,))
                    t0 = xb * w_v[b, r, pl.ds(i * D_OUT, LANES)]
                    t1 = xb * w_v[b, r, pl.ds(i * D_OUT + LANES, LANES)]
                    j = i % 4
                    acc0[j] = t0 if acc0[j] is None else acc0[j] + t0
                    acc1[j] = t1 if acc1[j] is None else acc1[j] + t1
                o_v[b, r, pl.ds(0, LANES)] = (
                    (acc0[0] + acc0[1]) + (acc0[2] + acc0[3]))
                o_v[b, r, pl.ds(LANES, LANES)] = (
                    (acc1[0] + acc1[1]) + (acc1[2] + acc1[3]))

            pltpu.sync_copy(o_v.at[b], out_hbm.at[pl.ds(row0, CHUNK), :])

        issue(0, 0)

        @pl.loop(0, n_chunks, step=N_BUF)
        def chunk_loop(k0):
            for b in range(N_BUF):
                k = k0 + b

                @pl.when(k + 1 < n_chunks)
                def _():
                    issue(k + 1, (b + 1) % N_BUF)

                compute(k, b)

    return seg_poly


@jax.jit
def kernel(weights, x, weight_indices):
    return _make_kernel(x.shape[0])(weights, x, weight_indices)
